# Initial kernel scaffold; baseline (speedup 1.0000x reference)
import jax
import jax.numpy as jnp
from jax.experimental import pallas as pl


def kernel(token_seq_inputs, embedding_table):
    raise NotImplementedError("write your pallas kernel here")


# speedup vs baseline: 2.7624x; 2.7624x over previous
import functools

import jax
import jax.numpy as jnp
from jax import lax
from jax.experimental import pallas as pl
from jax.experimental.pallas import tpu as pltpu
from jax.experimental.pallas import tpu_sc as plsc

_NUM_CORES = 2
_NUM_SUBCORES = 16
_NW = _NUM_CORES * _NUM_SUBCORES
_CHUNK = 128


@functools.lru_cache(maxsize=None)
def _build(n_rows: int, d: int):
    assert n_rows % (_NW * _CHUNK) == 0
    rows_per_w = n_rows // _NW
    n_chunks = rows_per_w // _CHUNK

    mesh = plsc.VectorSubcoreMesh(core_axis_name="c", subcore_axis_name="s")

    @functools.partial(
        pl.kernel,
        mesh=mesh,
        out_type=jax.ShapeDtypeStruct((n_rows, d), jnp.float32),
        scratch_types=[
            pltpu.VMEM((_CHUNK,), jnp.int32),
            pltpu.VMEM((_CHUNK, d), jnp.float32),
            pltpu.SemaphoreType.DMA,
        ],
    )
    def gather_kernel(idx_hbm, table_hbm, out_hbm, idx_v, rows_v, sem):
        wid = lax.axis_index("s") * _NUM_CORES + lax.axis_index("c")
        base = wid * rows_per_w

        def body(g, carry):
            off = base + g * _CHUNK
            pltpu.sync_copy(idx_hbm.at[pl.ds(off, _CHUNK)], idx_v)
            pltpu.async_copy(table_hbm.at[idx_v], rows_v, sem).wait()
            pltpu.sync_copy(rows_v, out_hbm.at[pl.ds(off, _CHUNK)])
            return carry

        lax.fori_loop(0, n_chunks, body, 0)

    return gather_kernel


def kernel(token_seq_inputs, embedding_table):
    batch, seq = token_seq_inputs.shape
    _, d = embedding_table.shape
    idx = token_seq_inputs.reshape(-1).astype(jnp.int32)
    n = idx.shape[0]
    pad = (-n) % (_NW * _CHUNK)
    if pad:
        idx = jnp.concatenate([idx, jnp.zeros((pad,), jnp.int32)])
    out = _build(n + pad, d)(idx, embedding_table)
    if pad:
        out = out[:n]
    return out.reshape(batch, seq, d)


# speedup vs baseline: 3.3394x; 1.2089x over previous
import functools

import jax
import jax.numpy as jnp
from jax import lax
from jax.experimental import pallas as pl
from jax.experimental.pallas import tpu as pltpu
from jax.experimental.pallas import tpu_sc as plsc

_NUM_CORES = 2
_NUM_SUBCORES = 16
_NW = _NUM_CORES * _NUM_SUBCORES
_CHUNK = 128
_NBUF = 5
_K = 3


@functools.lru_cache(maxsize=None)
def _build(n_rows: int, d: int):
    assert n_rows % (_NW * _CHUNK * _NBUF) == 0
    rows_per_w = n_rows // _NW
    n_chunks = rows_per_w // _CHUNK
    n_rounds = n_chunks // _NBUF
    assert n_rounds >= 2

    mesh = plsc.VectorSubcoreMesh(core_axis_name="c", subcore_axis_name="s")

    @functools.partial(
        pl.kernel,
        mesh=mesh,
        out_type=jax.ShapeDtypeStruct((n_rows, d), jnp.float32),
        scratch_types=[
            pltpu.VMEM((rows_per_w,), jnp.int32),
            pltpu.VMEM((_NBUF, _CHUNK, d), jnp.float32),
            pltpu.SemaphoreType.DMA((_NBUF,)),
            pltpu.SemaphoreType.DMA((_NBUF,)),
        ],
    )
    def gather_kernel(idx_hbm, table_hbm, out_hbm, idx_all, rows, gsem, osem):
        wid = lax.axis_index("s") * _NUM_CORES + lax.axis_index("c")
        base = wid * rows_per_w
        pltpu.sync_copy(idx_hbm.at[pl.ds(base, rows_per_w)], idx_all)

        def gather_copy(g, b):
            return pltpu.make_async_copy(
                table_hbm.at[idx_all.at[pl.ds(g * _CHUNK, _CHUNK)]],
                rows.at[b],
                gsem.at[b],
            )

        def out_copy(g, b):
            return pltpu.make_async_copy(
                rows.at[b],
                out_hbm.at[pl.ds(base + g * _CHUNK, _CHUNK)],
                osem.at[b],
            )

        def step(g, j, drain_out, issue_gather):
            gather_copy(g, j).wait()
            out_copy(g, j).start()
            if issue_gather:
                b2 = (j + _K) % _NBUF
                if drain_out:
                    out_copy(g, b2).wait()
                gather_copy(g + _K, b2).start()

        for b in range(_K):
            gather_copy(b, b).start()

        for j in range(_NBUF):
            step(j, j, drain_out=(j + _K >= _NBUF), issue_gather=True)

        def round_body(i, carry):
            g0 = i * _NBUF
            for j in range(_NBUF):
                step(g0 + j, j, drain_out=True, issue_gather=True)
            return carry

        lax.fori_loop(1, n_rounds - 1, round_body, 0)

        g0 = (n_rounds - 1) * _NBUF
        for j in range(_NBUF):
            g = g0 + j
            step(g, j, drain_out=True, issue_gather=(g + _K < n_chunks))

        for j in range(_NBUF):
            out_copy(g0 + j, j).wait()

    return gather_kernel


def kernel(token_seq_inputs, embedding_table):
    batch, seq = token_seq_inputs.shape
    _, d = embedding_table.shape
    idx = token_seq_inputs.reshape(-1).astype(jnp.int32)
    n = idx.shape[0]
    pad = (-n) % (_NW * _CHUNK * _NBUF)
    if pad:
        idx = jnp.concatenate([idx, jnp.zeros((pad,), jnp.int32)])
    out = _build(n + pad, d)(idx, embedding_table)
    if pad:
        out = out[:n]
    return out.reshape(batch, seq, d)


# speedup vs baseline: 3.3591x; 1.0059x over previous
import functools

import jax
import jax.numpy as jnp
from jax import lax
from jax.experimental import pallas as pl
from jax.experimental.pallas import tpu as pltpu
from jax.experimental.pallas import tpu_sc as plsc

_NUM_CORES = 2
_NUM_SUBCORES = 16
_NW = _NUM_CORES * _NUM_SUBCORES
_CHUNK = 64
_NBUF = 10
_K = 8


@functools.lru_cache(maxsize=None)
def _build(n_rows: int, d: int):
    assert n_rows % (_NW * _CHUNK * _NBUF) == 0
    rows_per_w = n_rows // _NW
    n_chunks = rows_per_w // _CHUNK
    n_rounds = n_chunks // _NBUF
    assert n_rounds >= 2

    mesh = plsc.VectorSubcoreMesh(core_axis_name="c", subcore_axis_name="s")

    @functools.partial(
        pl.kernel,
        mesh=mesh,
        out_type=jax.ShapeDtypeStruct((n_rows, d), jnp.float32),
        scratch_types=[
            pltpu.VMEM((rows_per_w,), jnp.int32),
            pltpu.VMEM((_NBUF, _CHUNK, d), jnp.float32),
            pltpu.SemaphoreType.DMA((_NBUF,)),
            pltpu.SemaphoreType.DMA((_NBUF,)),
        ],
    )
    def gather_kernel(idx_hbm, table_hbm, out_hbm, idx_all, rows, gsem, osem):
        wid = lax.axis_index("s") * _NUM_CORES + lax.axis_index("c")
        base = wid * rows_per_w
        pltpu.sync_copy(idx_hbm.at[pl.ds(base, rows_per_w)], idx_all)

        def gather_copy(g, b):
            return pltpu.make_async_copy(
                table_hbm.at[idx_all.at[pl.ds(g * _CHUNK, _CHUNK)]],
                rows.at[b],
                gsem.at[b],
            )

        def out_copy(g, b):
            return pltpu.make_async_copy(
                rows.at[b],
                out_hbm.at[pl.ds(base + g * _CHUNK, _CHUNK)],
                osem.at[b],
            )

        def step(g, j, drain_out, issue_gather):
            gather_copy(g, j).wait()
            out_copy(g, j).start()
            if issue_gather:
                b2 = (j + _K) % _NBUF
                if drain_out:
                    out_copy(g, b2).wait()
                gather_copy(g + _K, b2).start()

        for b in range(_K):
            gather_copy(b, b).start()

        for j in range(_NBUF):
            step(j, j, drain_out=(j + _K >= _NBUF), issue_gather=True)

        def round_body(i, carry):
            g0 = i * _NBUF
            for j in range(_NBUF):
                step(g0 + j, j, drain_out=True, issue_gather=True)
            return carry

        lax.fori_loop(1, n_rounds - 1, round_body, 0)

        g0 = (n_rounds - 1) * _NBUF
        for j in range(_NBUF):
            g = g0 + j
            step(g, j, drain_out=True, issue_gather=(g + _K < n_chunks))

        for j in range(_NBUF):
            out_copy(g0 + j, j).wait()

    return gather_kernel


def kernel(token_seq_inputs, embedding_table):
    batch, seq = token_seq_inputs.shape
    _, d = embedding_table.shape
    idx = token_seq_inputs.reshape(-1).astype(jnp.int32)
    n = idx.shape[0]
    pad = (-n) % (_NW * _CHUNK * _NBUF)
    if pad:
        idx = jnp.concatenate([idx, jnp.zeros((pad,), jnp.int32)])
    out = _build(n + pad, d)(idx, embedding_table)
    if pad:
        out = out[:n]
    return out.reshape(batch, seq, d)
